# baseline (speedup 1.0000x reference)
import functools

import jax
import jax.numpy as jnp
from jax import lax
from jax.experimental import pallas as pl
from jax.experimental.pallas import tpu as pltpu
from jax.experimental.pallas import tpu_sc as plsc

B = 16384
D = 32
VROWS = 250000
_NC = 2
_NS = 16
_NW = _NC * _NS
B_PER_W = B // _NW
CHUNK = 128
NCHUNK = B_PER_W // CHUNK
L = 16
NGROUP = B_PER_W // L


@functools.cache
def _build():
    mesh = plsc.VectorSubcoreMesh(core_axis_name="c", subcore_axis_name="s")

    @functools.partial(
        pl.kernel,
        mesh=mesh,
        out_type=jax.ShapeDtypeStruct((B, D), jnp.float32),
        scratch_types=[
            pltpu.VMEM((NCHUNK, CHUNK), jnp.int32),
            pltpu.VMEM((B_PER_W,), jnp.int32),
            pltpu.VMEM((B_PER_W, 128), jnp.float32),
            pltpu.VMEM((B_PER_W, D), jnp.float32),
            pltpu.SemaphoreType.DMA,
        ],
        compiler_params=pltpu.CompilerParams(
            use_tc_tiling_on_sc=False, needs_layout_passes=False),
    )
    def _gather_sc(hi_hbm, off_hbm, table_hbm, out_hbm,
                   hi_v, off_v, wide_v, out_v, sem):
        wid = lax.axis_index("s") * _NC + lax.axis_index("c")
        pltpu.sync_copy(hi_hbm.at[wid], hi_v)
        pltpu.sync_copy(off_hbm.at[wid], off_v)
        copies = [
            pltpu.async_copy(
                table_hbm.at[hi_v.at[j]],
                wide_v.at[pl.ds(j * CHUNK, CHUNK)],
                sem,
            )
            for j in range(NCHUNK)
        ]
        for c in copies:
            c.wait()
        iota = lax.iota(jnp.int32, L)

        def group_body(g, _):
            rowids = g * L + iota
            off = off_v[pl.ds(g * L, L)]
            for j in range(D):
                vals = plsc.load_gather(wide_v, [rowids, off + j])
                plsc.store_scatter(
                    out_v, [rowids, jnp.full((L,), j, jnp.int32)], vals)
            return _

        lax.fori_loop(0, NGROUP, group_body, 0)
        pltpu.sync_copy(out_v, out_hbm.at[pl.ds(wid * B_PER_W, B_PER_W)])

    return _gather_sc


def kernel(inputs, v):
    idx = inputs.reshape(_NW, NCHUNK, CHUNK).astype(jnp.int32)
    hi = idx >> 2
    off = ((idx & 3) << 5).reshape(_NW, B_PER_W)
    table = v.reshape(VROWS, 128)
    return _build()(hi, off, table)


# speedup vs baseline: 4.3930x; 4.3930x over previous
import functools

import jax
import jax.numpy as jnp
from jax import lax
from jax.experimental import pallas as pl
from jax.experimental.pallas import tpu as pltpu
from jax.experimental.pallas import tpu_sc as plsc

B = 16384
D = 32
V = 1000000
_NC = 2
_NS = 16
_NW = _NC * _NS
B_PER_W = B // _NW
NBUF = 8
NCH = B_PER_W // NBUF
L = 16
TILE_BYTES = D * 128 * 4


@functools.cache
def _build():
    mesh = plsc.VectorSubcoreMesh(core_axis_name="c", subcore_axis_name="s")

    @functools.partial(
        pl.kernel,
        mesh=mesh,
        out_type=jax.ShapeDtypeStruct((D, B), jnp.float32),
        scratch_types=[
            pltpu.SMEM((B_PER_W,), jnp.int32),
            pltpu.VMEM((B_PER_W,), jnp.int32),
            pltpu.VMEM((NBUF, D, 128), jnp.float32),
            pltpu.VMEM((D, B_PER_W), jnp.float32),
            [pltpu.SemaphoreType.DMA] * NBUF,
        ],
        compiler_params=pltpu.CompilerParams(
            use_tc_tiling_on_sc=True, needs_layout_passes=False),
    )
    def _gather_sc(idx_hbm, vt_hbm, out_hbm, idx_s, idx_v, bufs, cols_v, sems):
        wid = lax.axis_index("s") * _NC + lax.axis_index("c")
        base = pl.multiple_of(wid * B_PER_W, 128)
        pltpu.sync_copy(idx_hbm.at[pl.ds(base, B_PER_W)], idx_v)

        def spill(g, carry):
            vec = idx_v[pl.ds(g * L, L)]
            for k in range(L):
                idx_s[g * L + k] = vec[k]
            return carry

        lax.fori_loop(0, B_PER_W // L, spill, 0)

        iota = lax.iota(jnp.int32, L)
        rows0 = iota
        rows1 = iota + L

        def fire(i, b):
            c = idx_s[i] >> 7
            pltpu.async_copy(
                vt_hbm.at[:, pl.ds(pl.multiple_of(c * 128, 128), 128)],
                bufs.at[b],
                sems[b],
            )

        def wait(b):
            pltpu.make_async_copy(
                vt_hbm.at[:, pl.ds(0, 128)], bufs.at[b], sems[b]
            ).wait()

        def extract(i, b):
            lane = jnp.full((L,), idx_s[i] & 127, jnp.int32)
            col = jnp.full((L,), i, jnp.int32)
            v0 = plsc.load_gather(bufs.at[b], [rows0, lane])
            v1 = plsc.load_gather(bufs.at[b], [rows1, lane])
            plsc.store_scatter(cols_v, [rows0, col], v0)
            plsc.store_scatter(cols_v, [rows1, col], v1)

        for b in range(NBUF):
            fire(b, b)

        def chunk(g, carry):
            for b in range(NBUF):
                i = g * NBUF + b
                wait(b)
                extract(i, b)

                @pl.when(g < NCH - 1)
                def _():
                    fire(i + NBUF, b)

            return carry

        lax.fori_loop(0, NCH, chunk, 0)
        pltpu.sync_copy(cols_v, out_hbm.at[:, pl.ds(base, B_PER_W)])

    return _gather_sc


def kernel(inputs, v):
    idx = inputs.reshape(B).astype(jnp.int32)
    vt = jnp.transpose(v)
    out_t = _build()(idx, vt)
    return jnp.transpose(out_t)
